# baseline (speedup 1.0000x reference)
import jax
import jax.numpy as jnp
from jax.experimental import pallas as pl

_BB = 8


def _body(wt_ref, x_ref, o_ref):
    wt = wt_ref[...]
    for b in range(_BB):
        xb = x_ref[b]
        o_ref[b] = jax.lax.dot_general(
            wt, xb, (((1,), (1,)), ((), ())),
            preferred_element_type=jnp.float32)


def kernel(x, idx, vals, W):
    B, N, C = x.shape
    F = W.shape[1]
    wt = W.T
    return pl.pallas_call(
        _body,
        grid=(B // _BB,),
        in_specs=[
            pl.BlockSpec((F, C), lambda i: (0, 0)),
            pl.BlockSpec((_BB, N, C), lambda i: (i, 0, 0)),
        ],
        out_specs=pl.BlockSpec((_BB, F, N), lambda i: (i, 0, 0)),
        out_shape=jax.ShapeDtypeStruct((B, F, N), x.dtype),
    )(wt, x)


# speedup vs baseline: 1.0811x; 1.0811x over previous
import jax
import jax.numpy as jnp
from jax.experimental import pallas as pl

_BB = 16


def _body(w_ref, x_ref, o_ref):
    w = w_ref[...]
    for b in range(_BB):
        m = jax.lax.dot_general(
            x_ref[b], w, (((1,), (0,)), ((), ())),
            preferred_element_type=jnp.float32)
        o_ref[b] = m.T


def kernel(x, idx, vals, W):
    B, N, C = x.shape
    F = W.shape[1]
    return pl.pallas_call(
        _body,
        grid=(B // _BB,),
        in_specs=[
            pl.BlockSpec((C, F), lambda i: (0, 0)),
            pl.BlockSpec((_BB, N, C), lambda i: (i, 0, 0)),
        ],
        out_specs=pl.BlockSpec((_BB, F, N), lambda i: (i, 0, 0)),
        out_shape=jax.ShapeDtypeStruct((B, F, N), x.dtype),
    )(W, x)
